# baseline (speedup 1.0000x reference)
import functools

import jax
import jax.numpy as jnp
from jax import lax
from jax.experimental import pallas as pl
from jax.experimental.pallas import tpu as pltpu

_F32 = jnp.float32
_BF16 = jnp.bfloat16


def _lrelu(x):
    return jnp.maximum(x, x * x.dtype.type(0.01))


def _wd_kernel(
        xw_ref, xd_ref, xr_ref, len_ref,
        v2d_ref, v2dsq_ref, gmat_ref, waff_ref, baff_ref,
        w1_ref, b1_ref, w2_ref, b2_ref, w3_ref, b3_ref,
        wf_ref, bf_ref,
        wih0_ref, bi0_ref, whc_ref,
        gb0_ref, whh1_ref, gb1_ref, bi1n_ref,
        wrw_ref, wrd_ref, wrr_ref, br_ref,
        out_ref,
        *gi0_chunks,
        S, TB, H, CS):
    dot = lambda a, b: jnp.dot(a, b, preferred_element_type=_F32)
    H2 = 2 * H

    xw = xw_ref[...]
    xwb = xw.astype(_BF16)
    xwsq = (xw * xw).astype(_BF16)
    NW = v2d_ref.shape[1]
    NWh = NW // 2 if NW % 2 == 0 else NW
    cp = None
    for lo in range(0, NW, NWh):
        a = dot(xwb, v2d_ref[:, lo:lo + NWh])
        t2 = dot(xwsq, v2dsq_ref[:, lo:lo + NWh])
        term = (a * a - t2).astype(_BF16)
        piece = dot(term, gmat_ref[lo:lo + NWh, :])
        cp = piece if cp is None else cp + piece
    wide_out = dot(cp.astype(_BF16), waff_ref[...]) + baff_ref[...]

    h = _lrelu(dot(xd_ref[...].astype(_BF16), w1_ref[...]).astype(_BF16) + b1_ref[...])
    h = _lrelu(dot(h, w2_ref[...]).astype(_BF16) + b2_ref[...] + h)
    h = _lrelu(dot(h, w3_ref[...]).astype(_BF16) + b3_ref[...] + h)
    deep_out = h

    wf = wf_ref[...]
    bfb = bf_ref[...]
    wih0 = wih0_ref[...]
    bi0 = bi0_ref[...]

    def bulk_chunk(c):
        buf = gi0_chunks[c % len(gi0_chunks)]
        in_r = xr_ref.shape[2]
        PS = CS // 4 if CS % 4 == 0 else CS
        for p in range(0, CS, PS):
            t0 = c * CS + p
            xr = xr_ref[t0:t0 + PS].reshape(PS * TB, in_r).astype(_BF16)
            xt = _lrelu(dot(xr, wf).astype(_BF16) + bfb)
            buf[p * TB:(p + PS) * TB, :] = dot(xt, wih0).astype(_BF16) + bi0

    whc = whc_ref[...]
    whh1 = whh1_ref[...]
    gb0 = gb0_ref[...]
    gb1 = gb1_ref[...]
    bi1n = bi1n_ref[...]
    lens = len_ref[...]

    half = jnp.bfloat16(0.5)
    one = jnp.bfloat16(1.0)
    H3, H5 = 3 * H, 5 * H

    def l1_math(gi1, g1, h1, u):
        rz1 = half * jnp.tanh(gi1[:, :H2] + g1[:, :H2]) + half
        r1, z1 = rz1[:, :H], rz1[:, H:]
        n1 = jnp.tanh(gi1[:, H2:] + bi1n + r1 * g1[:, H2:])
        h1_new = n1 + z1 * (h1 - n1)
        return jnp.where(u < lens, h1_new, h1)

    def gi0_at(t):
        return gi0_chunks[(t // CS) % len(gi0_chunks)][(t % CS) * TB:(t % CS + 1) * TB, :]

    bulk_chunk(0)

    gi0 = gi0_at(0)
    rz0 = half * jnp.tanh(gi0[:, :H2]) + half
    z0 = rz0[:, H:]
    n0 = jnp.tanh(gi0[:, H2:] + rz0[:, :H] * gb0)
    h0 = (one - z0) * n0
    h1 = jnp.zeros((TB, H), _BF16)

    for t in range(1, S):
        if t % CS == 1 and t // CS + 1 < S // CS:
            bulk_chunk(t // CS + 1)
        g01 = dot(h0, whc).astype(_BF16)
        g1 = dot(h1, whh1).astype(_BF16) + gb1
        h1 = l1_math(g01[:, H3:], g1, h1, t - 1)
        gi0 = gi0_at(t)
        rz0 = half * jnp.tanh(gi0[:, :H2] + g01[:, :H2]) + half
        r0, z0 = rz0[:, :H], rz0[:, H:]
        n0 = jnp.tanh(gi0[:, H2:] + r0 * (g01[:, H2:H3] + gb0))
        h0 = n0 + z0 * (h0 - n0)
    gi1 = dot(h0, whc).astype(_BF16)[:, H3:]
    g1 = dot(h1, whh1).astype(_BF16) + gb1
    h1 = l1_math(gi1, g1, h1, S - 1)
    rnn_out = h1

    out_ref[...] = (dot(wide_out.astype(_BF16), wrw_ref[...])
                    + dot(deep_out.astype(_BF16), wrd_ref[...])
                    + dot(rnn_out, wrr_ref[...]) + br_ref[...])


def _pick_tb(batch):
    return batch


def kernel(V, Waff, baff, W1, b1, W2, b2, W3, b3, Wf, bf,
           Wih0, Whh0, bih0, bhh0, Wih1, Whh1, bih1, bhh1, Wr, br,
           x_wide, x_deep, x_rnn, lengths):
    Hw, in_w, k = V.shape
    S, B, in_r = x_rnn.shape
    H = Wf.shape[0]
    out_w = Waff.shape[0]
    hidden_d = W1.shape[0]
    out_d = W3.shape[0]
    in_d = x_deep.shape[1]

    TB = _pick_tb(B)
    nb = B // TB

    bcast = lambda w: w.astype(_BF16)

    v2d = jnp.transpose(V, (1, 0, 2)).reshape(in_w, Hw * k)
    v2d_sq = bcast(v2d * v2d)
    v2d = bcast(v2d)
    gmat = bcast(jnp.repeat(0.5 * jnp.eye(Hw, dtype=_F32), k, axis=0))

    half_rz = jnp.concatenate([jnp.full(2 * H, 0.5, _F32), jnp.ones(H, _F32)])[None, :]
    wih0_t = bcast(Wih0.T * half_rz)
    wih1_t = bcast(Wih1.T * half_rz)
    whh0_t = bcast(Whh0.T * half_rz)
    whh1_t = bcast(Whh1.T * half_rz)

    def gate_b(b):
        return b[:H], b[H:2 * H], b[2 * H:]

    bih0_r, bih0_z, bih0_n = gate_b(bih0)
    bhh0_r, bhh0_z, bhh0_n = gate_b(bhh0)
    bih1_r, bih1_z, bih1_n = gate_b(bih1)
    bhh1_r, bhh1_z, bhh1_n = gate_b(bhh1)
    bi0 = bcast(jnp.concatenate([bih0_r + bhh0_r, bih0_z + bhh0_z, bih0_n])[None, :]
                * half_rz)
    gb0 = bcast(bhh0_n[None, :])
    gb1 = bcast(jnp.concatenate([bih1_r + bhh1_r, bih1_z + bhh1_z, bhh1_n])[None, :]
                * half_rz)
    bi1n = bcast(bih1_n[None, :])

    wrw_t = bcast(Wr[:, :out_w].T)
    wrd_t = bcast(Wr[:, out_w:out_w + out_d].T)
    wrr_t = bcast(Wr[:, out_w + out_d:].T)

    lens2d = lengths.astype(jnp.int32).reshape(B, 1)

    args = (
        x_wide.astype(_F32), x_deep.astype(_F32), x_rnn.astype(_F32), lens2d,
        v2d, v2d_sq, gmat, bcast(Waff.T), baff[None, :].astype(_F32),
        bcast(W1.T), bcast(b1[None, :]),
        bcast(W2.T), bcast(b2[None, :]),
        bcast(W3.T), bcast(b3[None, :]),
        bcast(Wf.T), bcast(bf[None, :]),
        wih0_t, bi0, jnp.concatenate([whh0_t, wih1_t], axis=1),
        gb0, whh1_t, gb1, bi1n,
        wrw_t, wrd_t, wrr_t, br[None, :].astype(_F32),
    )

    bmap = lambda b: (b, 0)
    w2 = lambda b: (0, 0)
    in_specs = [
        pl.BlockSpec((TB, in_w), bmap),
        pl.BlockSpec((TB, in_d), bmap),
        pl.BlockSpec((S, TB, in_r), lambda b: (0, b, 0)),
        pl.BlockSpec((TB, 1), bmap),
        pl.BlockSpec((in_w, Hw * k), w2),
        pl.BlockSpec((in_w, Hw * k), w2),
        pl.BlockSpec((Hw * k, Hw), w2),
        pl.BlockSpec((Hw, out_w), w2),
        pl.BlockSpec((1, out_w), w2),
        pl.BlockSpec((in_d, hidden_d), w2),
        pl.BlockSpec((1, hidden_d), w2),
        pl.BlockSpec((hidden_d, hidden_d), w2),
        pl.BlockSpec((1, hidden_d), w2),
        pl.BlockSpec((hidden_d, out_d), w2),
        pl.BlockSpec((1, out_d), w2),
        pl.BlockSpec((in_r, H), w2),
        pl.BlockSpec((1, H), w2),
        pl.BlockSpec((H, 3 * H), w2),
        pl.BlockSpec((1, 3 * H), w2),
        pl.BlockSpec((H, 6 * H), w2),
        pl.BlockSpec((1, H), w2),
        pl.BlockSpec((H, 3 * H), w2),
        pl.BlockSpec((1, 3 * H), w2),
        pl.BlockSpec((1, H), w2),
        pl.BlockSpec((out_w, 1), w2),
        pl.BlockSpec((out_d, 1), w2),
        pl.BlockSpec((H, 1), w2),
        pl.BlockSpec((1, 1), w2),
    ]

    CS = 8 if S % 8 == 0 else S
    nc = min(S // CS, 2)
    _kernel_fn = functools.partial(_wd_kernel, S=S, TB=TB, H=H, CS=CS)
    return pl.pallas_call(
        _kernel_fn,
        out_shape=jax.ShapeDtypeStruct((B, 1), _F32),
        grid=(nb,),
        in_specs=in_specs,
        out_specs=pl.BlockSpec((TB, 1), bmap),
        scratch_shapes=[pltpu.VMEM((CS * TB, 3 * H), _BF16) for _ in range(nc)],
        compiler_params=pltpu.CompilerParams(dimension_semantics=("arbitrary",)),
    )(*args)


# speedup vs baseline: 1.1027x; 1.1027x over previous
import functools

import jax
import jax.numpy as jnp
from jax import lax
from jax.experimental import pallas as pl
from jax.experimental.pallas import tpu as pltpu

_F32 = jnp.float32
_BF16 = jnp.bfloat16


def _lrelu(x):
    return jnp.maximum(x, x * x.dtype.type(0.01))


def _wd_kernel(
        xw_ref, xd_ref, xr_ref, len_ref,
        v2d_ref, v2dsq_ref, gmat_ref, waff_ref, baff_ref,
        w1_ref, b1_ref, w2_ref, b2_ref, w3_ref, b3_ref,
        wf_ref, bf_ref,
        wih0_ref, bi0_ref, whc_ref,
        gb0_ref, whh1_ref, gb1_ref, bi1n_ref,
        wrw_ref, wrd_ref, wrr_ref, br_ref,
        out_ref,
        *gi0_chunks,
        S, TB, H, CS):
    dot = lambda a, b: jnp.dot(a, b, preferred_element_type=_F32)
    H2 = 2 * H

    xw = xw_ref[...]
    xwb = xw.astype(_BF16)
    xwsq = (xw * xw).astype(_BF16)
    NW = v2d_ref.shape[1]
    NWh = NW // 2 if NW % 2 == 0 else NW
    cp = None
    for lo in range(0, NW, NWh):
        a = dot(xwb, v2d_ref[:, lo:lo + NWh])
        t2 = dot(xwsq, v2dsq_ref[:, lo:lo + NWh])
        term = (a * a - t2).astype(_BF16)
        piece = dot(term, gmat_ref[lo:lo + NWh, :])
        cp = piece if cp is None else cp + piece
    wide_out = dot(cp.astype(_BF16), waff_ref[...]) + baff_ref[...]

    h = _lrelu(dot(xd_ref[...].astype(_BF16), w1_ref[...]).astype(_BF16) + b1_ref[...])
    h = _lrelu(dot(h, w2_ref[...]).astype(_BF16) + b2_ref[...] + h)
    h = _lrelu(dot(h, w3_ref[...]).astype(_BF16) + b3_ref[...] + h)
    deep_out = h

    wf = wf_ref[...]
    bfb = bf_ref[...]
    wih0 = wih0_ref[...]
    bi0 = bi0_ref[...]

    def bulk_chunk(c):
        buf = gi0_chunks[c % len(gi0_chunks)]
        in_r = xr_ref.shape[2]
        PS = CS // 4 if CS % 4 == 0 else CS
        for p in range(0, CS, PS):
            t0 = c * CS + p
            xr = xr_ref[t0:t0 + PS].reshape(PS * TB, in_r).astype(_BF16)
            xt = _lrelu(dot(xr, wf).astype(_BF16) + bfb)
            buf[p * TB:(p + PS) * TB, :] = dot(xt, wih0).astype(_BF16) + bi0

    whc = whc_ref[...]
    whh1 = whh1_ref[...]
    gb0 = gb0_ref[...]
    gb1 = gb1_ref[...]
    bi1n = bi1n_ref[...]
    lens = len_ref[...]

    half = jnp.bfloat16(0.5)
    one = jnp.bfloat16(1.0)
    H3, H5 = 3 * H, 5 * H

    def l1_math(gi1, g1, h1, u):
        rz1 = half * jnp.tanh(gi1[:, :H2] + g1[:, :H2]) + half
        r1, z1 = rz1[:, :H], rz1[:, H:]
        n1 = jnp.tanh(gi1[:, H2:] + bi1n + r1 * g1[:, H2:])
        h1_new = n1 + z1 * (h1 - n1)
        return jnp.where(u < lens, h1_new, h1)

    def gi0_at(t):
        return gi0_chunks[(t // CS) % len(gi0_chunks)][(t % CS) * TB:(t % CS + 1) * TB, :]

    bulk_chunk(0)

    gi0 = gi0_at(0)
    rz0 = half * jnp.tanh(gi0[:, :H2]) + half
    z0 = rz0[:, H:]
    n0 = jnp.tanh(gi0[:, H2:] + rz0[:, :H] * gb0)
    h0 = (one - z0) * n0
    h1 = jnp.zeros((TB, H), _BF16)

    for t in range(1, S):
        if t % CS == 1 and t // CS + 1 < S // CS:
            bulk_chunk(t // CS + 1)
        g01 = dot(h0, whc).astype(_BF16)
        g1 = dot(h1, whh1).astype(_BF16) + gb1
        h1 = l1_math(g01[:, H3:], g1, h1, t - 1)
        gi0 = gi0_at(t)
        rz0 = half * jnp.tanh(gi0[:, :H2] + g01[:, :H2]) + half
        r0, z0 = rz0[:, :H], rz0[:, H:]
        n0 = jnp.tanh(gi0[:, H2:] + r0 * (g01[:, H2:H3] + gb0))
        h0 = n0 + z0 * (h0 - n0)
    gi1 = dot(h0, whc).astype(_BF16)[:, H3:]
    g1 = dot(h1, whh1).astype(_BF16) + gb1
    h1 = l1_math(gi1, g1, h1, S - 1)
    rnn_out = h1

    out_ref[...] = (dot(wide_out.astype(_BF16), wrw_ref[...])
                    + dot(deep_out.astype(_BF16), wrd_ref[...])
                    + dot(rnn_out, wrr_ref[...]) + br_ref[...])


def _pick_tb(batch):
    if batch >= 16 and batch % 16 == 0:
        return batch // 2
    return batch


def kernel(V, Waff, baff, W1, b1, W2, b2, W3, b3, Wf, bf,
           Wih0, Whh0, bih0, bhh0, Wih1, Whh1, bih1, bhh1, Wr, br,
           x_wide, x_deep, x_rnn, lengths):
    Hw, in_w, k = V.shape
    S, B, in_r = x_rnn.shape
    H = Wf.shape[0]
    out_w = Waff.shape[0]
    hidden_d = W1.shape[0]
    out_d = W3.shape[0]
    in_d = x_deep.shape[1]

    TB = _pick_tb(B)
    nb = B // TB

    bcast = lambda w: w.astype(_BF16)

    v2d = jnp.transpose(V, (1, 0, 2)).reshape(in_w, Hw * k)
    v2d_sq = bcast(v2d * v2d)
    v2d = bcast(v2d)
    gmat = bcast(jnp.repeat(0.5 * jnp.eye(Hw, dtype=_F32), k, axis=0))

    half_rz = jnp.concatenate([jnp.full(2 * H, 0.5, _F32), jnp.ones(H, _F32)])[None, :]
    wih0_t = bcast(Wih0.T * half_rz)
    wih1_t = bcast(Wih1.T * half_rz)
    whh0_t = bcast(Whh0.T * half_rz)
    whh1_t = bcast(Whh1.T * half_rz)

    def gate_b(b):
        return b[:H], b[H:2 * H], b[2 * H:]

    bih0_r, bih0_z, bih0_n = gate_b(bih0)
    bhh0_r, bhh0_z, bhh0_n = gate_b(bhh0)
    bih1_r, bih1_z, bih1_n = gate_b(bih1)
    bhh1_r, bhh1_z, bhh1_n = gate_b(bhh1)
    bi0 = bcast(jnp.concatenate([bih0_r + bhh0_r, bih0_z + bhh0_z, bih0_n])[None, :]
                * half_rz)
    gb0 = bcast(bhh0_n[None, :])
    gb1 = bcast(jnp.concatenate([bih1_r + bhh1_r, bih1_z + bhh1_z, bhh1_n])[None, :]
                * half_rz)
    bi1n = bcast(bih1_n[None, :])

    wrw_t = bcast(Wr[:, :out_w].T)
    wrd_t = bcast(Wr[:, out_w:out_w + out_d].T)
    wrr_t = bcast(Wr[:, out_w + out_d:].T)

    lens2d = lengths.astype(jnp.int32).reshape(B, 1)

    args = (
        x_wide.astype(_F32), x_deep.astype(_F32), x_rnn.astype(_F32), lens2d,
        v2d, v2d_sq, gmat, bcast(Waff.T), baff[None, :].astype(_F32),
        bcast(W1.T), bcast(b1[None, :]),
        bcast(W2.T), bcast(b2[None, :]),
        bcast(W3.T), bcast(b3[None, :]),
        bcast(Wf.T), bcast(bf[None, :]),
        wih0_t, bi0, jnp.concatenate([whh0_t, wih1_t], axis=1),
        gb0, whh1_t, gb1, bi1n,
        wrw_t, wrd_t, wrr_t, br[None, :].astype(_F32),
    )

    bmap = lambda b: (b, 0)
    w2 = lambda b: (0, 0)
    in_specs = [
        pl.BlockSpec((TB, in_w), bmap),
        pl.BlockSpec((TB, in_d), bmap),
        pl.BlockSpec((S, TB, in_r), lambda b: (0, b, 0)),
        pl.BlockSpec((TB, 1), bmap),
        pl.BlockSpec((in_w, Hw * k), w2),
        pl.BlockSpec((in_w, Hw * k), w2),
        pl.BlockSpec((Hw * k, Hw), w2),
        pl.BlockSpec((Hw, out_w), w2),
        pl.BlockSpec((1, out_w), w2),
        pl.BlockSpec((in_d, hidden_d), w2),
        pl.BlockSpec((1, hidden_d), w2),
        pl.BlockSpec((hidden_d, hidden_d), w2),
        pl.BlockSpec((1, hidden_d), w2),
        pl.BlockSpec((hidden_d, out_d), w2),
        pl.BlockSpec((1, out_d), w2),
        pl.BlockSpec((in_r, H), w2),
        pl.BlockSpec((1, H), w2),
        pl.BlockSpec((H, 3 * H), w2),
        pl.BlockSpec((1, 3 * H), w2),
        pl.BlockSpec((H, 6 * H), w2),
        pl.BlockSpec((1, H), w2),
        pl.BlockSpec((H, 3 * H), w2),
        pl.BlockSpec((1, 3 * H), w2),
        pl.BlockSpec((1, H), w2),
        pl.BlockSpec((out_w, 1), w2),
        pl.BlockSpec((out_d, 1), w2),
        pl.BlockSpec((H, 1), w2),
        pl.BlockSpec((1, 1), w2),
    ]

    CS = 16 if S % 16 == 0 else S
    nc = S // CS
    _kernel_fn = functools.partial(_wd_kernel, S=S, TB=TB, H=H, CS=CS)
    return pl.pallas_call(
        _kernel_fn,
        out_shape=jax.ShapeDtypeStruct((B, 1), _F32),
        grid=(nb,),
        in_specs=in_specs,
        out_specs=pl.BlockSpec((TB, 1), bmap),
        scratch_shapes=[pltpu.VMEM((CS * TB, 3 * H), _BF16) for _ in range(nc)],
        compiler_params=pltpu.CompilerParams(dimension_semantics=("parallel",)),
    )(*args)
